# baseline (speedup 1.0000x reference)
import jax
import jax.numpy as jnp
from jax.experimental import pallas as pl
from jax.experimental.pallas import tpu as pltpu

_N = 16384
_C = 1000
_BLOCK_N = 1024
_SCALE = 1.0 / (224.0 * 224.0)


def _body(x_ref, t_ref, out_ref, acc_sum, acc_cnt):
    i = pl.program_id(0)

    first = i == 0

    def strip(r, carry):
        rows = pl.ds(r * 8, 8)

        def tile(j):
            xj = x_ref[rows, j * 128:(j + 1) * 128]
            tj = t_ref[rows, j * 128:(j + 1) * 128]
            dj = xj - tj
            mj = tj > 0.0
            return jnp.where(mj, dj * dj, 0.0), mj.astype(jnp.float32)

        ps, pc = tile(0)
        for j in range(1, _BLOCK_N // 128):
            vj, cj = tile(j)
            ps = ps + vj
            pc = pc + cj
        prev_s = jnp.where(first, 0.0, acc_sum[rows, :])
        prev_c = jnp.where(first, 0.0, acc_cnt[rows, :])
        acc_sum[rows, :] = prev_s + ps
        acc_cnt[rows, :] = prev_c + pc
        return carry

    jax.lax.fori_loop(0, _C // 8, strip, 0, unroll=False)

    @pl.when(i == pl.num_programs(0) - 1)
    def _fin():
        s = jnp.sum(acc_sum[...], axis=1, keepdims=True)
        n = jnp.sum(acc_cnt[...], axis=1, keepdims=True)
        mean = jnp.where(n > 0.0, s / jnp.maximum(n, 1.0), 0.0)
        out_ref[0, 0] = jnp.sum(mean) * _SCALE


def kernel(inputs, targets):
    x_t = inputs.T
    t_t = targets.T
    grid = (_N // _BLOCK_N,)
    out = pl.pallas_call(
        _body,
        grid=grid,
        in_specs=[
            pl.BlockSpec((_C, _BLOCK_N), lambda i: (0, i)),
            pl.BlockSpec((_C, _BLOCK_N), lambda i: (0, i)),
        ],
        out_specs=pl.BlockSpec(memory_space=pltpu.SMEM),
        out_shape=jax.ShapeDtypeStruct((1, 1), jnp.float32),
        compiler_params=pltpu.CompilerParams(
            vmem_limit_bytes=100 * 1024 * 1024,
        ),
        scratch_shapes=[
            pltpu.VMEM((_C, 128), jnp.float32),
            pltpu.VMEM((_C, 128), jnp.float32),
        ],
    )(x_t, t_t)
    return out[0, 0]


# speedup vs baseline: 1.1228x; 1.1228x over previous
import jax
import jax.numpy as jnp
from jax.experimental import pallas as pl
from jax.experimental.pallas import tpu as pltpu

_N = 16384
_C = 1000
_BLOCK_N = 2048
_SCALE = 1.0 / (224.0 * 224.0)


def _body(x_ref, t_ref, out_ref, acc_sum, acc_cnt):
    i = pl.program_id(0)

    first = i == 0

    def strip(r, carry):
        rows = pl.ds(r * 8, 8)

        def tile(j):
            xj = x_ref[rows, j * 128:(j + 1) * 128]
            tj = t_ref[rows, j * 128:(j + 1) * 128]
            dj = xj - tj
            mj = tj > 0.0
            return jnp.where(mj, dj * dj, 0.0), mj.astype(jnp.float32)

        parts = [tile(j) for j in range(_BLOCK_N // 128)]
        while len(parts) > 1:
            parts = [
                (a[0] + b[0], a[1] + b[1])
                for a, b in zip(parts[0::2], parts[1::2])
            ]
        ps, pc = parts[0]
        prev_s = jnp.where(first, 0.0, acc_sum[rows, :])
        prev_c = jnp.where(first, 0.0, acc_cnt[rows, :])
        acc_sum[rows, :] = prev_s + ps
        acc_cnt[rows, :] = prev_c + pc
        return carry

    jax.lax.fori_loop(0, _C // 8, strip, 0, unroll=False)

    @pl.when(i == pl.num_programs(0) - 1)
    def _fin():
        s = jnp.sum(acc_sum[...], axis=1, keepdims=True)
        n = jnp.sum(acc_cnt[...], axis=1, keepdims=True)
        mean = jnp.where(n > 0.0, s / jnp.maximum(n, 1.0), 0.0)
        out_ref[0, 0] = jnp.sum(mean) * _SCALE


def kernel(inputs, targets):
    x_t = inputs.T
    t_t = targets.T
    grid = (_N // _BLOCK_N,)
    out = pl.pallas_call(
        _body,
        grid=grid,
        in_specs=[
            pl.BlockSpec((_C, _BLOCK_N), lambda i: (0, i)),
            pl.BlockSpec((_C, _BLOCK_N), lambda i: (0, i)),
        ],
        out_specs=pl.BlockSpec(memory_space=pltpu.SMEM),
        out_shape=jax.ShapeDtypeStruct((1, 1), jnp.float32),
        compiler_params=pltpu.CompilerParams(
            vmem_limit_bytes=100 * 1024 * 1024,
        ),
        scratch_shapes=[
            pltpu.VMEM((_C, 128), jnp.float32),
            pltpu.VMEM((_C, 128), jnp.float32),
        ],
    )(x_t, t_t)
    return out[0, 0]


# speedup vs baseline: 1.1447x; 1.0195x over previous
import jax
import jax.numpy as jnp
from jax.experimental import pallas as pl
from jax.experimental.pallas import tpu as pltpu

_N = 16384
_C = 1000
_BLOCK_N = 2048
_SCALE = 1.0 / (224.0 * 224.0)


def _body(x_ref, t_ref, out_ref, acc_sum, acc_cnt):
    i = pl.program_id(0)

    first = i == 0

    def strip(r, carry):
        rows = pl.ds(r * 8, 8)

        def tile(j):
            xj = x_ref[rows, j * 128:(j + 1) * 128]
            tj = t_ref[rows, j * 128:(j + 1) * 128]
            dj = xj - tj
            mj = tj > 0.0
            return jnp.where(mj, dj * dj, 0.0), mj.astype(jnp.float32)

        parts = [tile(j) for j in range(_BLOCK_N // 128)]
        while len(parts) > 1:
            parts = [
                (a[0] + b[0], a[1] + b[1])
                for a, b in zip(parts[0::2], parts[1::2])
            ]
        ps, pc = parts[0]
        prev_s = jnp.where(first, 0.0, acc_sum[rows, :])
        prev_c = jnp.where(first, 0.0, acc_cnt[rows, :])
        acc_sum[rows, :] = prev_s + ps
        acc_cnt[rows, :] = prev_c + pc
        return carry

    jax.lax.fori_loop(0, _C // 8, strip, 0, unroll=4)

    @pl.when(i == pl.num_programs(0) - 1)
    def _fin():
        s = jnp.sum(acc_sum[...], axis=1, keepdims=True)
        n = jnp.sum(acc_cnt[...], axis=1, keepdims=True)
        mean = jnp.where(n > 0.0, s / jnp.maximum(n, 1.0), 0.0)
        out_ref[0, 0] = jnp.sum(mean) * _SCALE


def kernel(inputs, targets):
    x_t = inputs.T
    t_t = targets.T
    grid = (_N // _BLOCK_N,)
    out = pl.pallas_call(
        _body,
        grid=grid,
        in_specs=[
            pl.BlockSpec((_C, _BLOCK_N), lambda i: (0, i)),
            pl.BlockSpec((_C, _BLOCK_N), lambda i: (0, i)),
        ],
        out_specs=pl.BlockSpec(memory_space=pltpu.SMEM),
        out_shape=jax.ShapeDtypeStruct((1, 1), jnp.float32),
        compiler_params=pltpu.CompilerParams(
            vmem_limit_bytes=100 * 1024 * 1024,
        ),
        scratch_shapes=[
            pltpu.VMEM((_C, 128), jnp.float32),
            pltpu.VMEM((_C, 128), jnp.float32),
        ],
    )(x_t, t_t)
    return out[0, 0]


# speedup vs baseline: 1.1467x; 1.0018x over previous
import jax
import jax.numpy as jnp
from jax.experimental import pallas as pl
from jax.experimental.pallas import tpu as pltpu

_N = 16384
_C = 1000
_BLOCK_N = 2048
_SCALE = 1.0 / (224.0 * 224.0)


def _body(x_ref, t_ref, out_ref, acc_sum, acc_cnt):
    i = pl.program_id(0)

    first = i == 0

    def strip(r, carry):
        rows = pl.ds(r * 8, 8)

        def tile(j):
            xj = x_ref[rows, j * 128:(j + 1) * 128]
            tj = t_ref[rows, j * 128:(j + 1) * 128]
            dj = xj - tj
            mj = tj > 0.0
            return jnp.where(mj, dj * dj, 0.0), mj.astype(jnp.float32)

        parts = [tile(j) for j in range(_BLOCK_N // 128)]
        while len(parts) > 1:
            parts = [
                (a[0] + b[0], a[1] + b[1])
                for a, b in zip(parts[0::2], parts[1::2])
            ]
        ps, pc = parts[0]
        prev_s = jnp.where(first, 0.0, acc_sum[rows, :])
        prev_c = jnp.where(first, 0.0, acc_cnt[rows, :])
        acc_sum[rows, :] = prev_s + ps
        acc_cnt[rows, :] = prev_c + pc
        return carry

    jax.lax.fori_loop(0, _C // 8, strip, 0, unroll=8)

    @pl.when(i == pl.num_programs(0) - 1)
    def _fin():
        s = jnp.sum(acc_sum[...], axis=1, keepdims=True)
        n = jnp.sum(acc_cnt[...], axis=1, keepdims=True)
        mean = jnp.where(n > 0.0, s / jnp.maximum(n, 1.0), 0.0)
        out_ref[0, 0] = jnp.sum(mean) * _SCALE


def kernel(inputs, targets):
    x_t = inputs.T
    t_t = targets.T
    grid = (_N // _BLOCK_N,)
    out = pl.pallas_call(
        _body,
        grid=grid,
        in_specs=[
            pl.BlockSpec((_C, _BLOCK_N), lambda i: (0, i)),
            pl.BlockSpec((_C, _BLOCK_N), lambda i: (0, i)),
        ],
        out_specs=pl.BlockSpec(memory_space=pltpu.SMEM),
        out_shape=jax.ShapeDtypeStruct((1, 1), jnp.float32),
        compiler_params=pltpu.CompilerParams(
            vmem_limit_bytes=100 * 1024 * 1024,
        ),
        scratch_shapes=[
            pltpu.VMEM((_C, 128), jnp.float32),
            pltpu.VMEM((_C, 128), jnp.float32),
        ],
    )(x_t, t_t)
    return out[0, 0]


# speedup vs baseline: 1.1711x; 1.0213x over previous
import jax
import jax.numpy as jnp
from jax.experimental import pallas as pl
from jax.experimental.pallas import tpu as pltpu

_N = 16384
_C = 1000
_BLOCK_N = 2048
_SCALE = 1.0 / (224.0 * 224.0)


def _body(x_ref, t_ref, out_ref, acc_sum, acc_cnt):
    i = pl.program_id(0)

    first = i == 0

    def strip(r, carry):
        rows = pl.ds(r * 8, 8)

        def tile(j):
            xj = x_ref[rows, j * 128:(j + 1) * 128]
            tj = t_ref[rows, j * 128:(j + 1) * 128]
            dj = xj - tj
            mj = tj > 0.0
            return jnp.where(mj, dj * dj, 0.0), mj.astype(jnp.float32)

        parts = [tile(j) for j in range(_BLOCK_N // 128)]
        while len(parts) > 1:
            parts = [
                (a[0] + b[0], a[1] + b[1])
                for a, b in zip(parts[0::2], parts[1::2])
            ]
        ps, pc = parts[0]
        prev_s = jnp.where(first, 0.0, acc_sum[rows, :])
        prev_c = jnp.where(first, 0.0, acc_cnt[rows, :])
        acc_sum[rows, :] = prev_s + ps
        acc_cnt[rows, :] = prev_c + pc
        return carry

    jax.lax.fori_loop(0, _C // 8, strip, 0, unroll=16)

    @pl.when(i == pl.num_programs(0) - 1)
    def _fin():
        s = jnp.sum(acc_sum[...], axis=1, keepdims=True)
        n = jnp.sum(acc_cnt[...], axis=1, keepdims=True)
        mean = jnp.where(n > 0.0, s / jnp.maximum(n, 1.0), 0.0)
        out_ref[0, 0] = jnp.sum(mean) * _SCALE


def kernel(inputs, targets):
    x_t = inputs.T
    t_t = targets.T
    grid = (_N // _BLOCK_N,)
    out = pl.pallas_call(
        _body,
        grid=grid,
        in_specs=[
            pl.BlockSpec((_C, _BLOCK_N), lambda i: (0, i)),
            pl.BlockSpec((_C, _BLOCK_N), lambda i: (0, i)),
        ],
        out_specs=pl.BlockSpec(memory_space=pltpu.SMEM),
        out_shape=jax.ShapeDtypeStruct((1, 1), jnp.float32),
        compiler_params=pltpu.CompilerParams(
            vmem_limit_bytes=100 * 1024 * 1024,
        ),
        scratch_shapes=[
            pltpu.VMEM((_C, 128), jnp.float32),
            pltpu.VMEM((_C, 128), jnp.float32),
        ],
    )(x_t, t_t)
    return out[0, 0]
